# baseline (speedup 1.0000x reference)
import jax
import jax.numpy as jnp
from jax.experimental import pallas as pl
from jax.experimental.pallas import tpu as pltpu

_DN_T = (((1,), (1,)), ((), ()))


def _fused_mlp(x_ref, w1_ref, b1_ref, w2_ref, b2_ref, w3_ref, b3_ref,
               wc_ref, f_ref, s_ref):
    h = jax.lax.dot_general(x_ref[...], w1_ref[...], _DN_T,
                            preferred_element_type=jnp.float32)
    h = jnp.maximum(h + b1_ref[...], 0.0)
    h = jax.lax.dot_general(h, w2_ref[...], _DN_T,
                            preferred_element_type=jnp.float32)
    h = jnp.maximum(h + b2_ref[...], 0.0)
    f = jax.lax.dot_general(h, w3_ref[...], _DN_T,
                            preferred_element_type=jnp.float32)
    f = jnp.maximum(f + b3_ref[...], 0.0)
    f_ref[...] = f
    s_ref[...] = jax.lax.dot_general(f, wc_ref[...], _DN_T,
                                     preferred_element_type=jnp.float32)


def kernel(x, W1, b1, W2, b2, W3, b3, Wc):
    N, D = x.shape
    H1 = W1.shape[0]
    H2 = W2.shape[0]
    H3 = W3.shape[0]
    C = Wc.shape[0]

    BN = 16384

    features, scores = pl.pallas_call(
        _fused_mlp,
        grid=(N // BN,),
        compiler_params=pltpu.CompilerParams(
            dimension_semantics=("parallel",),
        ),
        in_specs=[
            pl.BlockSpec((BN, D), lambda i: (i, 0)),
            pl.BlockSpec((H1, D), lambda i: (0, 0)),
            pl.BlockSpec((H1,), lambda i: (0,)),
            pl.BlockSpec((H2, H1), lambda i: (0, 0)),
            pl.BlockSpec((H2,), lambda i: (0,)),
            pl.BlockSpec((H3, H2), lambda i: (0, 0)),
            pl.BlockSpec((H3,), lambda i: (0,)),
            pl.BlockSpec((C, H3), lambda i: (0, 0)),
        ],
        out_specs=[
            pl.BlockSpec((BN, H3), lambda i: (i, 0)),
            pl.BlockSpec((BN, C), lambda i: (i, 0)),
        ],
        out_shape=[
            jax.ShapeDtypeStruct((N, H3), jnp.float32),
            jax.ShapeDtypeStruct((N, C), jnp.float32),
        ],
    )(x, W1, b1, W2, b2, W3, b3, Wc)
    return (features, scores)


# speedup vs baseline: 1.0597x; 1.0597x over previous
import jax
import jax.numpy as jnp
from jax.experimental import pallas as pl
from jax.experimental.pallas import tpu as pltpu

_DN_T = (((1,), (1,)), ((), ()))
_S = 4


def _fused_mlp(x0_ref, x1_ref, x2_ref, x3_ref, w1_ref, b1_ref, w2_ref,
               b2_ref, w3_ref, b3_ref, wc_ref, f_ref, s_ref):
    sub = x0_ref.shape[0]
    for j, x_ref in enumerate((x0_ref, x1_ref, x2_ref, x3_ref)):
        h = jax.lax.dot_general(x_ref[...], w1_ref[...], _DN_T,
                                preferred_element_type=jnp.float32)
        h = jnp.maximum(h + b1_ref[...], 0.0)
        h = jax.lax.dot_general(h, w2_ref[...], _DN_T,
                                preferred_element_type=jnp.float32)
        h = jnp.maximum(h + b2_ref[...], 0.0)
        f = jax.lax.dot_general(h, w3_ref[...], _DN_T,
                                preferred_element_type=jnp.float32)
        f = jnp.maximum(f + b3_ref[...], 0.0)
        f_ref[pl.ds(j * sub, sub), :] = f
        s_ref[pl.ds(j * sub, sub), :] = jax.lax.dot_general(
            f, wc_ref[...], _DN_T, preferred_element_type=jnp.float32)


def kernel(x, W1, b1, W2, b2, W3, b3, Wc):
    N, D = x.shape
    H1 = W1.shape[0]
    H2 = W2.shape[0]
    H3 = W3.shape[0]
    C = Wc.shape[0]

    BN = 4096
    SUB = BN // _S

    def _x_spec(j):
        return pl.BlockSpec((SUB, D), lambda i, j=j: (_S * i + j, 0))

    features, scores = pl.pallas_call(
        _fused_mlp,
        grid=(N // BN,),
        compiler_params=pltpu.CompilerParams(
            dimension_semantics=("parallel",),
        ),
        in_specs=[
            _x_spec(0), _x_spec(1), _x_spec(2), _x_spec(3),
            pl.BlockSpec((H1, D), lambda i: (0, 0)),
            pl.BlockSpec((H1,), lambda i: (0,)),
            pl.BlockSpec((H2, H1), lambda i: (0, 0)),
            pl.BlockSpec((H2,), lambda i: (0,)),
            pl.BlockSpec((H3, H2), lambda i: (0, 0)),
            pl.BlockSpec((H3,), lambda i: (0,)),
            pl.BlockSpec((C, H3), lambda i: (0, 0)),
        ],
        out_specs=[
            pl.BlockSpec((BN, H3), lambda i: (i, 0)),
            pl.BlockSpec((BN, C), lambda i: (i, 0)),
        ],
        out_shape=[
            jax.ShapeDtypeStruct((N, H3), jnp.float32),
            jax.ShapeDtypeStruct((N, C), jnp.float32),
        ],
    )(x, x, x, x, W1, b1, W2, b2, W3, b3, Wc)
    return (features, scores)
